# Initial kernel scaffold; baseline (speedup 1.0000x reference)
import functools

import jax
import jax.numpy as jnp
from jax import lax
from jax.experimental import pallas as pl
from jax.experimental.pallas import tpu as pltpu
from jax.experimental.pallas import tpu_sc as plsc

N = 10000
E = 160000
D = 256
H = 256
O = 128
G = 64

NC = 2
NS = 16
L = 16

B = 128
EP = 163840
ER = EP // B
NACC = 10016
TRASH = 10008
STRIPE = N // NS
ZSTRIPE = NACC // NS

BLK = 1000
NB = N // BLK

_MESH = plsc.VectorSubcoreMesh(core_axis_name="c", subcore_axis_name="s")


def _sc_degree(dst2d):
    rpt = ER // (NC * NS)

    @functools.partial(
        pl.kernel,
        mesh=_MESH,
        out_type=[jax.ShapeDtypeStruct((N, L), jnp.float32),
                  jax.ShapeDtypeStruct((N, L), jnp.float32)],
        scratch_types=[pltpu.VMEM((rpt, B), jnp.int32),
                       pltpu.VMEM((B, L), jnp.float32),
                       pltpu.VMEM((ZSTRIPE, L), jnp.float32),
                       pltpu.VMEM_SHARED((NACC, L), jnp.float32)],
    )
    def k(dst_hbm, deg0_hbm, deg1_hbm, idx_v, ones_v, zero_v, acc):
        c = lax.axis_index("c")
        s = lax.axis_index("s")
        w = c * NS + s

        @pl.loop(0, B)
        def _(i):
            ones_v[i, :] = jnp.ones((L,), jnp.float32)

        @pl.loop(0, ZSTRIPE)
        def _(i):
            zero_v[i, :] = jnp.zeros((L,), jnp.float32)

        pltpu.sync_copy(dst_hbm.at[pl.ds(w * rpt, rpt)], idx_v)
        pltpu.sync_copy(zero_v, acc.at[pl.ds(s * ZSTRIPE, ZSTRIPE)])
        plsc.subcore_barrier()

        @pl.loop(0, rpt)
        def _(j):
            pltpu.sync_copy(ones_v, acc.at[idx_v.at[j]], add=True)

        plsc.subcore_barrier()

        @pl.when(c == 0)
        def _():
            pltpu.sync_copy(acc.at[pl.ds(s * STRIPE, STRIPE)],
                            deg0_hbm.at[pl.ds(s * STRIPE, STRIPE)])

        @pl.when(c == 1)
        def _():
            pltpu.sync_copy(acc.at[pl.ds(s * STRIPE, STRIPE)],
                            deg1_hbm.at[pl.ds(s * STRIPE, STRIPE)])

    return k(dst2d)


def _sc_scatter(y0, y1, src2d, dst2d):
    rpt = ER // NS

    @functools.partial(
        pl.kernel,
        mesh=_MESH,
        out_type=[jax.ShapeDtypeStruct((N, 128), jnp.float32),
                  jax.ShapeDtypeStruct((N, 128), jnp.float32)],
        scratch_types=[pltpu.VMEM((rpt, B), jnp.int32),
                       pltpu.VMEM((rpt, B), jnp.int32),
                       pltpu.VMEM((B, 128), jnp.float32),
                       pltpu.VMEM_SHARED((NACC, 128), jnp.float32)],
    )
    def k(y0_hbm, y1_hbm, src2d_hbm, dst2d_hbm, s0_hbm, s1_hbm,
          src_v, dst_v, rows_v, acc):
        c = lax.axis_index("c")
        s = lax.axis_index("s")

        pltpu.sync_copy(src2d_hbm.at[pl.ds(s * rpt, rpt)], src_v)
        pltpu.sync_copy(dst2d_hbm.at[pl.ds(s * rpt, rpt)], dst_v)

        def run(y_hbm, out_hbm):
            pltpu.sync_copy(y_hbm.at[pl.ds(s * STRIPE, STRIPE)],
                            acc.at[pl.ds(s * STRIPE, STRIPE)])
            plsc.subcore_barrier()

            @pl.loop(0, rpt)
            def _(j):
                pltpu.sync_copy(y_hbm.at[src_v.at[j]], rows_v)
                pltpu.sync_copy(rows_v, acc.at[dst_v.at[j]], add=True)

            plsc.subcore_barrier()
            pltpu.sync_copy(acc.at[pl.ds(s * STRIPE, STRIPE)],
                            out_hbm.at[pl.ds(s * STRIPE, STRIPE)])

        @pl.when(c == 0)
        def _():
            run(y0_hbm, s0_hbm)

        @pl.when(c == 1)
        def _():
            run(y1_hbm, s1_hbm)

    return k(y0, y1, src2d, dst2d)


def _tc_xw(x, W):
    def body(x_ref, w_ref, o0_ref, o1_ref):
        xw = jnp.dot(x_ref[...], w_ref[...], preferred_element_type=jnp.float32)
        o0_ref[...] = xw[:, :128]
        o1_ref[...] = xw[:, 128:]

    return pl.pallas_call(
        body,
        grid=(NB,),
        in_specs=[pl.BlockSpec((BLK, D), lambda i: (i, 0)),
                  pl.BlockSpec((D, H), lambda i: (0, 0))],
        out_specs=[pl.BlockSpec((BLK, 128), lambda i: (i, 0)),
                   pl.BlockSpec((BLK, 128), lambda i: (i, 0))],
        out_shape=[jax.ShapeDtypeStruct((N, 128), jnp.float32),
                   jax.ShapeDtypeStruct((N, 128), jnp.float32)],
    )(x, W)


def _tc_scale(xw0, xw1, deg0, deg1):
    def body(a0, a1, d0, d1, y0, y1, dv):
        dsum = d0[...] + d1[...] + 1.0
        dinv = lax.rsqrt(jnp.maximum(dsum[:, :1], 1e-12))
        dvb = jnp.broadcast_to(dinv, (BLK, 128))
        dv[...] = dvb
        y0[...] = a0[...] * dvb
        y1[...] = a1[...] * dvb

    return pl.pallas_call(
        body,
        grid=(NB,),
        in_specs=[pl.BlockSpec((BLK, 128), lambda i: (i, 0)),
                  pl.BlockSpec((BLK, 128), lambda i: (i, 0)),
                  pl.BlockSpec((BLK, L), lambda i: (i, 0)),
                  pl.BlockSpec((BLK, L), lambda i: (i, 0))],
        out_specs=[pl.BlockSpec((BLK, 128), lambda i: (i, 0)),
                   pl.BlockSpec((BLK, 128), lambda i: (i, 0)),
                   pl.BlockSpec((BLK, 128), lambda i: (i, 0))],
        out_shape=[jax.ShapeDtypeStruct((N, 128), jnp.float32),
                   jax.ShapeDtypeStruct((N, 128), jnp.float32),
                   jax.ShapeDtypeStruct((N, 128), jnp.float32)],
    )(xw0, xw1, deg0, deg1)


def _tc_layer2(S0, S1, dv, b1, W2):
    def body(s0, s1, d, b, w, y0, y1):
        dvb = d[...]
        h0 = jnp.maximum(s0[...] * dvb + b[...][:, :128], 0.0)
        h1 = jnp.maximum(s1[...] * dvb + b[...][:, 128:], 0.0)
        yw = (jnp.dot(h0, w[...][:128, :], preferred_element_type=jnp.float32)
              + jnp.dot(h1, w[...][128:, :], preferred_element_type=jnp.float32))
        y0[...] = yw[:, :128] * dvb
        y1[...] = yw[:, 128:] * dvb

    return pl.pallas_call(
        body,
        grid=(NB,),
        in_specs=[pl.BlockSpec((BLK, 128), lambda i: (i, 0)),
                  pl.BlockSpec((BLK, 128), lambda i: (i, 0)),
                  pl.BlockSpec((BLK, 128), lambda i: (i, 0)),
                  pl.BlockSpec((1, H), lambda i: (0, 0)),
                  pl.BlockSpec((H, H), lambda i: (0, 0))],
        out_specs=[pl.BlockSpec((BLK, 128), lambda i: (i, 0)),
                   pl.BlockSpec((BLK, 128), lambda i: (i, 0))],
        out_shape=[jax.ShapeDtypeStruct((N, 128), jnp.float32),
                   jax.ShapeDtypeStruct((N, 128), jnp.float32)],
    )(S0, S1, dv, b1, W2)


def _tc_pool_fc(S0, S1, dv, b2, batch3, Wfc, bfc):
    def body(s0, s1, d, b, bt, w, bf, o_ref, pool_acc, cnt_acc):
        i = pl.program_id(0)

        @pl.when(i == 0)
        def _():
            pool_acc[...] = jnp.zeros_like(pool_acc)
            cnt_acc[...] = jnp.zeros_like(cnt_acc)

        dvb = d[...]
        h0 = jnp.maximum(s0[...] * dvb + b[...][:, :128], 0.0)
        h1 = jnp.maximum(s1[...] * dvb + b[...][:, 128:], 0.0)
        bt_row = bt[...].reshape(1, BLK)
        gid = lax.broadcasted_iota(jnp.int32, (G, BLK), 0)
        oh = (gid == jnp.broadcast_to(bt_row, (G, BLK))).astype(jnp.float32)
        pool_acc[:, :128] += jnp.dot(oh, h0, preferred_element_type=jnp.float32)
        pool_acc[:, 128:] += jnp.dot(oh, h1, preferred_element_type=jnp.float32)
        cnt_acc[...] += jnp.broadcast_to(
            jnp.sum(oh, axis=1, keepdims=True), (G, 128))

        @pl.when(i == NB - 1)
        def _():
            pooled = pool_acc[...] / jnp.maximum(cnt_acc[...][:, :1], 1.0)
            logits = jnp.dot(pooled, w[...],
                             preferred_element_type=jnp.float32) + bf[...]
            o_ref[...] = jax.nn.sigmoid(logits)

    return pl.pallas_call(
        body,
        grid=(NB,),
        in_specs=[pl.BlockSpec((BLK, 128), lambda i: (i, 0)),
                  pl.BlockSpec((BLK, 128), lambda i: (i, 0)),
                  pl.BlockSpec((BLK, 128), lambda i: (i, 0)),
                  pl.BlockSpec((1, H), lambda i: (0, 0)),
                  pl.BlockSpec((1, 1, BLK), lambda i: (i, 0, 0)),
                  pl.BlockSpec((H, O), lambda i: (0, 0)),
                  pl.BlockSpec((1, O), lambda i: (0, 0))],
        out_specs=pl.BlockSpec((G, O), lambda i: (0, 0)),
        out_shape=jax.ShapeDtypeStruct((G, O), jnp.float32),
        scratch_shapes=[pltpu.VMEM((G, H), jnp.float32),
                        pltpu.VMEM((G, 128), jnp.float32)],
    )(S0, S1, dv, b2, batch3, Wfc, bfc)


def kernel(x, edge_index, batch, W1, b1, W2, b2, Wfc, bfc):
    src = edge_index[0]
    dst = edge_index[1]
    pad = EP - E
    srcp = jnp.concatenate(
        [src, jnp.zeros((pad,), jnp.int32)]).reshape(ER, B)
    dstp = jnp.concatenate(
        [dst, jnp.full((pad,), TRASH, jnp.int32)]).reshape(ER, B)

    deg0, deg1 = _sc_degree(dstp)
    xw0, xw1 = _tc_xw(x, W1)
    y0, y1, dv = _tc_scale(xw0, xw1, deg0, deg1)
    S0, S1 = _sc_scatter(y0, y1, srcp, dstp)
    y20, y21 = _tc_layer2(S0, S1, dv, b1.reshape(1, H), W2)
    T0, T1 = _sc_scatter(y20, y21, srcp, dstp)
    out = _tc_pool_fc(T0, T1, dv, b2.reshape(1, H),
                      batch.reshape(NB, 1, BLK), Wfc, bfc.reshape(1, O))
    return out


# speedup vs baseline: 7.1357x; 7.1357x over previous
import functools

import jax
import jax.numpy as jnp
from jax import lax
from jax.experimental import pallas as pl
from jax.experimental.pallas import tpu as pltpu
from jax.experimental.pallas import tpu_sc as plsc

N = 10000
E = 160000
D = 256
H = 256
O = 128
G = 64

NC = 2
NS = 16
L = 16

B = 128
EP = 163840
ER = EP // B
NROW = 10240
TRASH = 10008
STRIPE = NROW // NS
CH = 128
CH128 = 32
IB = 8

BLK = 1000
NB = N // BLK

_MESH = plsc.VectorSubcoreMesh(core_axis_name="c", subcore_axis_name="s")


def _sc_degree(dst2d):
    rpt = ER // (NC * NS)

    @functools.partial(
        pl.kernel,
        mesh=_MESH,
        out_type=[jax.ShapeDtypeStruct((NROW, 128), jnp.float32),
                  jax.ShapeDtypeStruct((NROW, 128), jnp.float32)],
        scratch_types=[pltpu.VMEM((IB, B), jnp.int32),
                       pltpu.VMEM((B, 128), jnp.float32),
                       pltpu.VMEM_SHARED((NROW, 128), jnp.float32)],
    )
    def k(dst_hbm, deg0_hbm, deg1_hbm, idx_v, ones_v, acc):
        c = lax.axis_index("c")
        s = lax.axis_index("s")
        w = c * NS + s

        @pl.loop(0, B)
        def _(i):
            for q in range(128 // L):
                ones_v[i, pl.ds(q * L, L)] = jnp.ones((L,), jnp.float32)

        @pl.loop(0, STRIPE // CH128)
        def _(t):
            pltpu.sync_copy(ones_v.at[pl.ds(0, CH128)],
                            acc.at[pl.ds(s * STRIPE + t * CH128, CH128)])

        plsc.subcore_barrier()

        @pl.loop(0, rpt // IB)
        def _(t):
            pltpu.sync_copy(dst_hbm.at[pl.ds(w * rpt + t * IB, IB)], idx_v)
            for r in range(IB):
                pltpu.sync_copy(ones_v, acc.at[idx_v.at[r]], add=True)

        plsc.subcore_barrier()

        def write(deg_hbm):
            @pl.loop(0, STRIPE // CH128)
            def _(t):
                o = s * STRIPE + t * CH128
                pltpu.sync_copy(acc.at[pl.ds(o, CH128)],
                                deg_hbm.at[pl.ds(o, CH128)])

        @pl.when(c == 0)
        def _():
            write(deg0_hbm)

        @pl.when(c == 1)
        def _():
            write(deg1_hbm)

    return k(dst2d)


def _sc_scatter(y0, y1, src2d, dst2d):
    rpt = ER // NS

    @functools.partial(
        pl.kernel,
        mesh=_MESH,
        out_type=[jax.ShapeDtypeStruct((NROW, 128), jnp.float32),
                  jax.ShapeDtypeStruct((NROW, 128), jnp.float32)],
        scratch_types=[pltpu.VMEM((IB, B), jnp.int32),
                       pltpu.VMEM((IB, B), jnp.int32),
                       pltpu.VMEM((B, 128), jnp.float32),
                       pltpu.VMEM_SHARED((NROW, 128), jnp.float32)],
    )
    def k(y0_hbm, y1_hbm, src2d_hbm, dst2d_hbm, s0_hbm, s1_hbm,
          src_v, dst_v, rows_v, acc):
        c = lax.axis_index("c")
        s = lax.axis_index("s")

        def run(y_hbm, out_hbm):
            @pl.loop(0, STRIPE // CH128)
            def _(t):
                o = s * STRIPE + t * CH128
                pltpu.sync_copy(y_hbm.at[pl.ds(o, CH128)],
                                acc.at[pl.ds(o, CH128)])

            plsc.subcore_barrier()

            @pl.loop(0, rpt // IB)
            def _(t):
                pltpu.sync_copy(src2d_hbm.at[pl.ds(s * rpt + t * IB, IB)],
                                src_v)
                pltpu.sync_copy(dst2d_hbm.at[pl.ds(s * rpt + t * IB, IB)],
                                dst_v)
                for r in range(IB):
                    pltpu.sync_copy(y_hbm.at[src_v.at[r]], rows_v)
                    pltpu.sync_copy(rows_v, acc.at[dst_v.at[r]], add=True)

            plsc.subcore_barrier()

            @pl.loop(0, STRIPE // CH128)
            def _(t):
                o = s * STRIPE + t * CH128
                pltpu.sync_copy(acc.at[pl.ds(o, CH128)],
                                out_hbm.at[pl.ds(o, CH128)])

        @pl.when(c == 0)
        def _():
            run(y0_hbm, s0_hbm)

        @pl.when(c == 1)
        def _():
            run(y1_hbm, s1_hbm)

    return k(y0, y1, src2d, dst2d)


def _tc_xw(x, W):
    def body(x_ref, w_ref, o0_ref, o1_ref):
        xw = jnp.dot(x_ref[...], w_ref[...], preferred_element_type=jnp.float32)
        o0_ref[...] = xw[:, :128]
        o1_ref[...] = xw[:, 128:]

    return pl.pallas_call(
        body,
        grid=(NB,),
        in_specs=[pl.BlockSpec((BLK, D), lambda i: (i, 0)),
                  pl.BlockSpec((D, H), lambda i: (0, 0))],
        out_specs=[pl.BlockSpec((BLK, 128), lambda i: (i, 0)),
                   pl.BlockSpec((BLK, 128), lambda i: (i, 0))],
        out_shape=[jax.ShapeDtypeStruct((N, 128), jnp.float32),
                   jax.ShapeDtypeStruct((N, 128), jnp.float32)],
    )(x, W)


def _tc_scale(xw0, xw1, deg0, deg1):
    def body(a0, a1, d0, d1, y0, y1, dv):
        dsum = d0[...][:, :1] + d1[...][:, :1] - 1.0
        dinv = lax.rsqrt(jnp.maximum(dsum, 1e-12))
        dvb = jnp.broadcast_to(dinv, (BLK, 128))
        dv[...] = dvb
        y0[...] = a0[...] * dvb
        y1[...] = a1[...] * dvb

    return pl.pallas_call(
        body,
        grid=(NB,),
        in_specs=[pl.BlockSpec((BLK, 128), lambda i: (i, 0)),
                  pl.BlockSpec((BLK, 128), lambda i: (i, 0)),
                  pl.BlockSpec((BLK, 128), lambda i: (i, 0)),
                  pl.BlockSpec((BLK, 128), lambda i: (i, 0))],
        out_specs=[pl.BlockSpec((BLK, 128), lambda i: (i, 0)),
                   pl.BlockSpec((BLK, 128), lambda i: (i, 0)),
                   pl.BlockSpec((BLK, 128), lambda i: (i, 0))],
        out_shape=[jax.ShapeDtypeStruct((NROW, 128), jnp.float32),
                   jax.ShapeDtypeStruct((NROW, 128), jnp.float32),
                   jax.ShapeDtypeStruct((N, 128), jnp.float32)],
    )(xw0, xw1, deg0, deg1)


def _tc_layer2(S0, S1, dv, b1, W2):
    def body(s0, s1, d, b, w, y0, y1):
        dvb = d[...]
        h0 = jnp.maximum(s0[...] * dvb + b[...][:, :128], 0.0)
        h1 = jnp.maximum(s1[...] * dvb + b[...][:, 128:], 0.0)
        yw = (jnp.dot(h0, w[...][:128, :], preferred_element_type=jnp.float32)
              + jnp.dot(h1, w[...][128:, :], preferred_element_type=jnp.float32))
        y0[...] = yw[:, :128] * dvb
        y1[...] = yw[:, 128:] * dvb

    return pl.pallas_call(
        body,
        grid=(NB,),
        in_specs=[pl.BlockSpec((BLK, 128), lambda i: (i, 0)),
                  pl.BlockSpec((BLK, 128), lambda i: (i, 0)),
                  pl.BlockSpec((BLK, 128), lambda i: (i, 0)),
                  pl.BlockSpec((1, H), lambda i: (0, 0)),
                  pl.BlockSpec((H, H), lambda i: (0, 0))],
        out_specs=[pl.BlockSpec((BLK, 128), lambda i: (i, 0)),
                   pl.BlockSpec((BLK, 128), lambda i: (i, 0))],
        out_shape=[jax.ShapeDtypeStruct((NROW, 128), jnp.float32),
                   jax.ShapeDtypeStruct((NROW, 128), jnp.float32)],
    )(S0, S1, dv, b1, W2)


def _tc_pool_fc(S0, S1, dv, b2, batch3, Wfc, bfc):
    def body(s0, s1, d, b, bt, w, bf, o_ref, pool_acc, cnt_acc):
        i = pl.program_id(0)

        @pl.when(i == 0)
        def _():
            pool_acc[...] = jnp.zeros_like(pool_acc)
            cnt_acc[...] = jnp.zeros_like(cnt_acc)

        dvb = d[...]
        h0 = jnp.maximum(s0[...] * dvb + b[...][:, :128], 0.0)
        h1 = jnp.maximum(s1[...] * dvb + b[...][:, 128:], 0.0)
        bt_row = bt[...].reshape(1, BLK)
        gid = lax.broadcasted_iota(jnp.int32, (G, BLK), 0)
        oh = (gid == jnp.broadcast_to(bt_row, (G, BLK))).astype(jnp.float32)
        pool_acc[:, :128] += jnp.dot(oh, h0, preferred_element_type=jnp.float32)
        pool_acc[:, 128:] += jnp.dot(oh, h1, preferred_element_type=jnp.float32)
        cnt_acc[...] += jnp.broadcast_to(
            jnp.sum(oh, axis=1, keepdims=True), (G, 128))

        @pl.when(i == NB - 1)
        def _():
            pooled = pool_acc[...] / jnp.maximum(cnt_acc[...][:, :1], 1.0)
            logits = jnp.dot(pooled, w[...],
                             preferred_element_type=jnp.float32) + bf[...]
            o_ref[...] = jax.nn.sigmoid(logits)

    return pl.pallas_call(
        body,
        grid=(NB,),
        in_specs=[pl.BlockSpec((BLK, 128), lambda i: (i, 0)),
                  pl.BlockSpec((BLK, 128), lambda i: (i, 0)),
                  pl.BlockSpec((BLK, 128), lambda i: (i, 0)),
                  pl.BlockSpec((1, H), lambda i: (0, 0)),
                  pl.BlockSpec((1, 1, BLK), lambda i: (i, 0, 0)),
                  pl.BlockSpec((H, O), lambda i: (0, 0)),
                  pl.BlockSpec((1, O), lambda i: (0, 0))],
        out_specs=pl.BlockSpec((G, O), lambda i: (0, 0)),
        out_shape=jax.ShapeDtypeStruct((G, O), jnp.float32),
        scratch_shapes=[pltpu.VMEM((G, H), jnp.float32),
                        pltpu.VMEM((G, 128), jnp.float32)],
    )(S0, S1, dv, b2, batch3, Wfc, bfc)


def kernel(x, edge_index, batch, W1, b1, W2, b2, Wfc, bfc):
    src = edge_index[0]
    dst = edge_index[1]
    pad = EP - E
    srcp = jnp.concatenate(
        [src, jnp.zeros((pad,), jnp.int32)]).reshape(ER, B)
    dstp = jnp.concatenate(
        [dst, jnp.full((pad,), TRASH, jnp.int32)]).reshape(ER, B)

    deg0, deg1 = _sc_degree(dstp)
    xw0, xw1 = _tc_xw(x, W1)
    y0, y1, dv = _tc_scale(xw0, xw1, deg0, deg1)
    S0, S1 = _sc_scatter(y0, y1, srcp, dstp)
    y20, y21 = _tc_layer2(S0, S1, dv, b1.reshape(1, H), W2)
    T0, T1 = _sc_scatter(y20, y21, srcp, dstp)
    out = _tc_pool_fc(T0, T1, dv, b2.reshape(1, H),
                      batch.reshape(NB, 1, BLK), Wfc, bfc.reshape(1, O))
    return out


# speedup vs baseline: 7.8170x; 1.0955x over previous
import functools

import jax
import jax.numpy as jnp
from jax import lax
from jax.experimental import pallas as pl
from jax.experimental.pallas import tpu as pltpu
from jax.experimental.pallas import tpu_sc as plsc

N = 10000
E = 160000
D = 256
H = 256
O = 128
G = 64

NC = 2
NS = 16
L = 16

B = 128
EP = 163840
ER = EP // B
NROW = 10240
TRASH = 10008
STRIPE = NROW // NS
CH = 128
CH128 = 32
IB = 8

BLK = 1000
NB = N // BLK

_MESH = plsc.VectorSubcoreMesh(core_axis_name="c", subcore_axis_name="s")


def _sc_degree(dst2d):
    rpt = ER // (NC * NS)

    @functools.partial(
        pl.kernel,
        mesh=_MESH,
        out_type=[jax.ShapeDtypeStruct((NROW, 128), jnp.float32),
                  jax.ShapeDtypeStruct((NROW, 128), jnp.float32)],
        scratch_types=[pltpu.VMEM((IB, B), jnp.int32),
                       pltpu.VMEM((B, 128), jnp.float32),
                       pltpu.VMEM_SHARED((NROW, 128), jnp.float32)],
    )
    def k(dst_hbm, deg0_hbm, deg1_hbm, idx_v, ones_v, acc):
        c = lax.axis_index("c")
        s = lax.axis_index("s")
        w = c * NS + s

        @pl.loop(0, B)
        def _(i):
            for q in range(128 // L):
                ones_v[i, pl.ds(q * L, L)] = jnp.ones((L,), jnp.float32)

        @pl.loop(0, STRIPE // CH128)
        def _(t):
            pltpu.sync_copy(ones_v.at[pl.ds(0, CH128)],
                            acc.at[pl.ds(s * STRIPE + t * CH128, CH128)])

        plsc.subcore_barrier()

        @pl.loop(0, rpt // IB)
        def _(t):
            pltpu.sync_copy(dst_hbm.at[pl.ds(w * rpt + t * IB, IB)], idx_v)
            for r in range(IB):
                pltpu.sync_copy(ones_v, acc.at[idx_v.at[r]], add=True)

        plsc.subcore_barrier()

        def write(deg_hbm):
            @pl.loop(0, STRIPE // CH128)
            def _(t):
                o = s * STRIPE + t * CH128
                pltpu.sync_copy(acc.at[pl.ds(o, CH128)],
                                deg_hbm.at[pl.ds(o, CH128)])

        @pl.when(c == 0)
        def _():
            write(deg0_hbm)

        @pl.when(c == 1)
        def _():
            write(deg1_hbm)

    return k(dst2d)


def _sc_scatter(y0, y1, src2d, dst2d):
    rpt = ER // NS

    @functools.partial(
        pl.kernel,
        mesh=_MESH,
        out_type=[jax.ShapeDtypeStruct((NROW, 128), jnp.float32),
                  jax.ShapeDtypeStruct((NROW, 128), jnp.float32)],
        scratch_types=[pltpu.VMEM((IB, B), jnp.int32),
                       pltpu.VMEM((IB, B), jnp.int32),
                       pltpu.VMEM((B, 128), jnp.float32),
                       pltpu.VMEM((B, 128), jnp.float32),
                       pltpu.VMEM_SHARED((NROW, 128), jnp.float32),
                       pltpu.SemaphoreType.DMA,
                       pltpu.SemaphoreType.DMA],
    )
    def k(y0_hbm, y1_hbm, src2d_hbm, dst2d_hbm, s0_hbm, s1_hbm,
          src_v, dst_v, rb0, rb1, acc, sem0, sem1):
        c = lax.axis_index("c")
        s = lax.axis_index("s")

        def run(y_hbm, out_hbm):
            @pl.loop(0, STRIPE // CH128)
            def _(t):
                o = s * STRIPE + t * CH128
                pltpu.sync_copy(y_hbm.at[pl.ds(o, CH128)],
                                acc.at[pl.ds(o, CH128)])

            plsc.subcore_barrier()

            @pl.loop(0, rpt // IB)
            def _(t):
                pltpu.sync_copy(src2d_hbm.at[pl.ds(s * rpt + t * IB, IB)],
                                src_v)
                pltpu.sync_copy(dst2d_hbm.at[pl.ds(s * rpt + t * IB, IB)],
                                dst_v)
                pltpu.make_async_copy(y_hbm.at[src_v.at[0]], rb0,
                                      sem0).start()
                for r in range(IB):
                    rb, sem = (rb0, sem0) if r % 2 == 0 else (rb1, sem1)
                    pltpu.make_async_copy(y_hbm.at[src_v.at[r]], rb,
                                          sem).wait()
                    if r + 1 < IB:
                        nrb, nsem = (rb1, sem1) if r % 2 == 0 else (rb0, sem0)
                        pltpu.make_async_copy(y_hbm.at[src_v.at[r + 1]], nrb,
                                              nsem).start()
                    pltpu.sync_copy(rb, acc.at[dst_v.at[r]], add=True)

            plsc.subcore_barrier()

            @pl.loop(0, STRIPE // CH128)
            def _(t):
                o = s * STRIPE + t * CH128
                pltpu.sync_copy(acc.at[pl.ds(o, CH128)],
                                out_hbm.at[pl.ds(o, CH128)])

        @pl.when(c == 0)
        def _():
            run(y0_hbm, s0_hbm)

        @pl.when(c == 1)
        def _():
            run(y1_hbm, s1_hbm)

    return k(y0, y1, src2d, dst2d)


def _tc_xw(x, W):
    def body(x_ref, w_ref, o0_ref, o1_ref):
        xw = jnp.dot(x_ref[...], w_ref[...], preferred_element_type=jnp.float32)
        o0_ref[...] = xw[:, :128]
        o1_ref[...] = xw[:, 128:]

    return pl.pallas_call(
        body,
        grid=(NB,),
        in_specs=[pl.BlockSpec((BLK, D), lambda i: (i, 0)),
                  pl.BlockSpec((D, H), lambda i: (0, 0))],
        out_specs=[pl.BlockSpec((BLK, 128), lambda i: (i, 0)),
                   pl.BlockSpec((BLK, 128), lambda i: (i, 0))],
        out_shape=[jax.ShapeDtypeStruct((N, 128), jnp.float32),
                   jax.ShapeDtypeStruct((N, 128), jnp.float32)],
    )(x, W)


def _tc_scale(xw0, xw1, deg0, deg1):
    def body(a0, a1, d0, d1, y0, y1, dv):
        dsum = d0[...][:, :1] + d1[...][:, :1] - 1.0
        dinv = lax.rsqrt(jnp.maximum(dsum, 1e-12))
        dvb = jnp.broadcast_to(dinv, (BLK, 128))
        dv[...] = dvb
        y0[...] = a0[...] * dvb
        y1[...] = a1[...] * dvb

    return pl.pallas_call(
        body,
        grid=(NB,),
        in_specs=[pl.BlockSpec((BLK, 128), lambda i: (i, 0)),
                  pl.BlockSpec((BLK, 128), lambda i: (i, 0)),
                  pl.BlockSpec((BLK, 128), lambda i: (i, 0)),
                  pl.BlockSpec((BLK, 128), lambda i: (i, 0))],
        out_specs=[pl.BlockSpec((BLK, 128), lambda i: (i, 0)),
                   pl.BlockSpec((BLK, 128), lambda i: (i, 0)),
                   pl.BlockSpec((BLK, 128), lambda i: (i, 0))],
        out_shape=[jax.ShapeDtypeStruct((NROW, 128), jnp.float32),
                   jax.ShapeDtypeStruct((NROW, 128), jnp.float32),
                   jax.ShapeDtypeStruct((N, 128), jnp.float32)],
    )(xw0, xw1, deg0, deg1)


def _tc_layer2(S0, S1, dv, b1, W2):
    def body(s0, s1, d, b, w, y0, y1):
        dvb = d[...]
        h0 = jnp.maximum(s0[...] * dvb + b[...][:, :128], 0.0)
        h1 = jnp.maximum(s1[...] * dvb + b[...][:, 128:], 0.0)
        yw = (jnp.dot(h0, w[...][:128, :], preferred_element_type=jnp.float32)
              + jnp.dot(h1, w[...][128:, :], preferred_element_type=jnp.float32))
        y0[...] = yw[:, :128] * dvb
        y1[...] = yw[:, 128:] * dvb

    return pl.pallas_call(
        body,
        grid=(NB,),
        in_specs=[pl.BlockSpec((BLK, 128), lambda i: (i, 0)),
                  pl.BlockSpec((BLK, 128), lambda i: (i, 0)),
                  pl.BlockSpec((BLK, 128), lambda i: (i, 0)),
                  pl.BlockSpec((1, H), lambda i: (0, 0)),
                  pl.BlockSpec((H, H), lambda i: (0, 0))],
        out_specs=[pl.BlockSpec((BLK, 128), lambda i: (i, 0)),
                   pl.BlockSpec((BLK, 128), lambda i: (i, 0))],
        out_shape=[jax.ShapeDtypeStruct((NROW, 128), jnp.float32),
                   jax.ShapeDtypeStruct((NROW, 128), jnp.float32)],
    )(S0, S1, dv, b1, W2)


def _tc_pool_fc(S0, S1, dv, b2, batch3, Wfc, bfc):
    def body(s0, s1, d, b, bt, w, bf, o_ref, pool_acc, cnt_acc):
        i = pl.program_id(0)

        @pl.when(i == 0)
        def _():
            pool_acc[...] = jnp.zeros_like(pool_acc)
            cnt_acc[...] = jnp.zeros_like(cnt_acc)

        dvb = d[...]
        h0 = jnp.maximum(s0[...] * dvb + b[...][:, :128], 0.0)
        h1 = jnp.maximum(s1[...] * dvb + b[...][:, 128:], 0.0)
        bt_row = bt[...].reshape(1, BLK)
        gid = lax.broadcasted_iota(jnp.int32, (G, BLK), 0)
        oh = (gid == jnp.broadcast_to(bt_row, (G, BLK))).astype(jnp.float32)
        pool_acc[:, :128] += jnp.dot(oh, h0, preferred_element_type=jnp.float32)
        pool_acc[:, 128:] += jnp.dot(oh, h1, preferred_element_type=jnp.float32)
        cnt_acc[...] += jnp.broadcast_to(
            jnp.sum(oh, axis=1, keepdims=True), (G, 128))

        @pl.when(i == NB - 1)
        def _():
            pooled = pool_acc[...] / jnp.maximum(cnt_acc[...][:, :1], 1.0)
            logits = jnp.dot(pooled, w[...],
                             preferred_element_type=jnp.float32) + bf[...]
            o_ref[...] = jax.nn.sigmoid(logits)

    return pl.pallas_call(
        body,
        grid=(NB,),
        in_specs=[pl.BlockSpec((BLK, 128), lambda i: (i, 0)),
                  pl.BlockSpec((BLK, 128), lambda i: (i, 0)),
                  pl.BlockSpec((BLK, 128), lambda i: (i, 0)),
                  pl.BlockSpec((1, H), lambda i: (0, 0)),
                  pl.BlockSpec((1, 1, BLK), lambda i: (i, 0, 0)),
                  pl.BlockSpec((H, O), lambda i: (0, 0)),
                  pl.BlockSpec((1, O), lambda i: (0, 0))],
        out_specs=pl.BlockSpec((G, O), lambda i: (0, 0)),
        out_shape=jax.ShapeDtypeStruct((G, O), jnp.float32),
        scratch_shapes=[pltpu.VMEM((G, H), jnp.float32),
                        pltpu.VMEM((G, 128), jnp.float32)],
    )(S0, S1, dv, b2, batch3, Wfc, bfc)


def kernel(x, edge_index, batch, W1, b1, W2, b2, Wfc, bfc):
    src = edge_index[0]
    dst = edge_index[1]
    pad = EP - E
    srcp = jnp.concatenate(
        [src, jnp.zeros((pad,), jnp.int32)]).reshape(ER, B)
    dstp = jnp.concatenate(
        [dst, jnp.full((pad,), TRASH, jnp.int32)]).reshape(ER, B)

    deg0, deg1 = _sc_degree(dstp)
    xw0, xw1 = _tc_xw(x, W1)
    y0, y1, dv = _tc_scale(xw0, xw1, deg0, deg1)
    S0, S1 = _sc_scatter(y0, y1, srcp, dstp)
    y20, y21 = _tc_layer2(S0, S1, dv, b1.reshape(1, H), W2)
    T0, T1 = _sc_scatter(y20, y21, srcp, dstp)
    out = _tc_pool_fc(T0, T1, dv, b2.reshape(1, H),
                      batch.reshape(NB, 1, BLK), Wfc, bfc.reshape(1, O))
    return out
